# Initial kernel scaffold; baseline (speedup 1.0000x reference)
import jax
import jax.numpy as jnp
from jax.experimental import pallas as pl


def kernel(x, tok_table, pos_table):
    raise NotImplementedError("write your pallas kernel here")


# speedup vs baseline: 1.5981x; 1.5981x over previous
import jax
import jax.numpy as jnp
from jax import lax
from jax.experimental import pallas as pl
from jax.experimental.pallas import tpu as pltpu
from jax.experimental.pallas import tpu_sc as plsc

_B = 4
_S = 2048
_D = 1024
_NC = 2
_NS = 16
_NW = _NC * _NS
_SPW = _S // _NW
_CH = 16
_NJ = _SPW // _CH
_NT = _B * _NJ
_NGBUF = 3
_NOBUF = 2
_VPC = _CH * _D // 16


def _body(x_hbm, tok_hbm, pos_hbm, out_hbm,
          idx_v, p0, p1, g0, g1, g2, o0, o1,
          psem0, psem1, gsem0, gsem1, gsem2, osem0, osem1):
    pbufs = (p0, p1)
    gbufs = (g0, g1, g2)
    obufs = (o0, o1)
    psems = (psem0, psem1)
    gsems = (gsem0, gsem1, gsem2)
    osems = (osem0, osem1)

    wid = lax.axis_index("s") * _NC + lax.axis_index("c")
    s0 = wid * _SPW

    for b in range(_B):
        pltpu.sync_copy(x_hbm.at[pl.ds(b * _S + s0, _SPW)],
                        idx_v.at[pl.ds(b * _SPW, _SPW)])

    def start_pos(j):
        return pltpu.async_copy(
            pos_hbm.at[pl.ds(s0 + j * _CH, _CH)], pbufs[j % 2], psems[j % 2])

    def start_gather(t):
        j, b = t // _B, t % _B
        off = b * _SPW + j * _CH
        return pltpu.async_copy(
            tok_hbm.at[idx_v.at[pl.ds(off, _CH)]], gbufs[t % _NGBUF],
            gsems[t % _NGBUF])

    ph = [start_pos(0), None]
    gh = [None] * _NT
    for t in range(_NGBUF):
        gh[t] = start_gather(t)
    oh = [None, None]

    for j in range(_NJ):
        if j + 1 < _NJ:
            ph[(j + 1) % 2] = start_pos(j + 1)
        ph[j % 2].wait()
        pbuf = pbufs[j % 2]
        for b in range(_B):
            t = j * _B + b
            gbuf = gbufs[t % _NGBUF]
            obuf = obufs[t % _NOBUF]
            gh[t].wait()
            if oh[t % _NOBUF] is not None:
                oh[t % _NOBUF].wait()

            @plsc.parallel_loop(0, _VPC, unroll=4)
            def _(i, gbuf=gbuf, obuf=obuf, pbuf=pbuf):
                r = i >> 6
                c = pl.multiple_of((i & 63) << 4, 16)
                obuf[r, pl.ds(c, 16)] = (gbuf[r, pl.ds(c, 16)]
                                         + pbuf[r, pl.ds(c, 16)])

            flat = b * _S + s0 + j * _CH
            oh[t % _NOBUF] = pltpu.async_copy(
                obuf, out_hbm.at[pl.ds(flat, _CH)], osems[t % _NOBUF])
            if t + _NGBUF < _NT:
                gh[t + _NGBUF] = start_gather(t + _NGBUF)

    oh[0].wait()
    oh[1].wait()


def kernel(x, tok_table, pos_table):
    x_flat = x.reshape(-1).astype(jnp.int32)
    mesh = plsc.VectorSubcoreMesh(core_axis_name="c", subcore_axis_name="s")
    out = pl.kernel(
        _body,
        out_type=jax.ShapeDtypeStruct((_B * _S, _D), jnp.float32),
        mesh=mesh,
        scratch_types=[
            pltpu.VMEM((_B * _SPW,), jnp.int32),
            pltpu.VMEM((_CH, _D), jnp.float32),
            pltpu.VMEM((_CH, _D), jnp.float32),
            pltpu.VMEM((_CH, _D), jnp.float32),
            pltpu.VMEM((_CH, _D), jnp.float32),
            pltpu.VMEM((_CH, _D), jnp.float32),
            pltpu.VMEM((_CH, _D), jnp.float32),
            pltpu.VMEM((_CH, _D), jnp.float32),
            pltpu.SemaphoreType.DMA,
            pltpu.SemaphoreType.DMA,
            pltpu.SemaphoreType.DMA,
            pltpu.SemaphoreType.DMA,
            pltpu.SemaphoreType.DMA,
            pltpu.SemaphoreType.DMA,
            pltpu.SemaphoreType.DMA,
        ],
    )(x_flat, tok_table, pos_table)
    return out.reshape(_B, _S, _D)
